# baseline (speedup 1.0000x reference)
import functools

import jax
import jax.numpy as jnp
from jax import lax
from jax.experimental import pallas as pl
from jax.experimental.pallas import tpu as pltpu
from jax.experimental.pallas import tpu_sc as plsc

N = 10000
NP = 10240
E = 160000
EP = 163840
IN = 256
HID = 512
OUT = 128
K = 32
PAD_NODE = 10200

BM = 512
GRID_M = NP // BM
NSUB = 16
NCORE = 2
EROWS = EP // 128
DEG_ROWS = EROWS // (NSUB * NCORE)
AGG_ROWS = EROWS // NSUB
SLAB = NP // NSUB


def _deg_body(srcR, dstR, zbins, out, sv, dv, bins_s, bins_d):
    c = lax.axis_index("c")
    s = lax.axis_index("s")
    w = s * NCORE + c
    pltpu.sync_copy(srcR.at[pl.ds(w * DEG_ROWS, DEG_ROWS)], sv)
    pltpu.sync_copy(dstR.at[pl.ds(w * DEG_ROWS, DEG_ROWS)], dv)
    pltpu.sync_copy(zbins, bins_s)
    pltpu.sync_copy(zbins, bins_d)
    ones = jnp.ones((16,), jnp.float32)

    def step(j, carry):
        r = j // 8
        off = (j % 8) * 16
        plsc.addupdate_scatter(bins_s, [sv[r, pl.ds(off, 16)]], ones)
        plsc.addupdate_scatter(bins_d, [dv[r, pl.ds(off, 16)]], ones)
        return carry

    lax.fori_loop(0, DEG_ROWS * 8, step, 0)
    pltpu.sync_copy(bins_s, out.at[w, 0])
    pltpu.sync_copy(bins_d, out.at[w, 1])


def _deg_pallas(srcR, dstR, zbins):
    mesh = plsc.VectorSubcoreMesh(core_axis_name="c", subcore_axis_name="s")
    f = pl.kernel(
        _deg_body,
        out_type=jax.ShapeDtypeStruct((NCORE * NSUB, 2, NP), jnp.float32),
        mesh=mesh,
        compiler_params=pltpu.CompilerParams(needs_layout_passes=False),
        scratch_types=[
            pltpu.VMEM((DEG_ROWS, 128), jnp.int32),
            pltpu.VMEM((DEG_ROWS, 128), jnp.int32),
            pltpu.VMEM((NP,), jnp.float32),
            pltpu.VMEM((NP,), jnp.float32),
        ],
    )
    return f(srcR, dstR, zbins)


NBUF = 2
UNIT = 128
UNITS = (AGG_ROWS * 128) // UNIT


def _extract(packed_v, svring, dvring, u, unit):
    for i in range(UNIT // 16):
        off = unit * UNIT + i * 16
        p = packed_v[off // 128, pl.ds(off % 128, 16)]
        svring[u, pl.ds(i * 16, 16)] = jnp.bitwise_and(p, 0xFFFF)
        dvring[u, pl.ds(i * 16, 16)] = lax.shift_right_logical(p, 16)


def _agg_chunk(s_hbm, out_hbm, zrows, packed_v, svring, dvring,
               bufs, gsems, ssems, acc, s):
    pltpu.sync_copy(zrows, acc.at[pl.ds(s * SLAB, SLAB)])
    plsc.subcore_barrier()
    for u in range(NBUF):
        _extract(packed_v, svring, dvring, u, u)
        pltpu.async_copy(s_hbm.at[svring.at[u]], bufs[u], gsems[u])

    def step(i, carry):
        j = i * NBUF
        for u in range(NBUF):
            jj = j + u
            pltpu.make_async_copy(s_hbm.at[svring.at[u]], bufs[u],
                                  gsems[u]).wait()
            pltpu.async_copy(bufs[u], acc.at[dvring.at[u]], ssems[u],
                             add=True)

            @pl.when(jj + NBUF < UNITS)
            def _(u=u, jj=jj):
                pltpu.make_async_copy(bufs[u], acc.at[dvring.at[u]],
                                      ssems[u]).wait()
                _extract(packed_v, svring, dvring, u, jj + NBUF)
                pltpu.async_copy(s_hbm.at[svring.at[u]], bufs[u],
                                 gsems[u])
        return carry

    lax.fori_loop(0, UNITS // NBUF, step, 0)
    for u in range(NBUF):
        pltpu.make_async_copy(bufs[u], acc.at[dvring.at[u]],
                              ssems[u]).wait()
    plsc.subcore_barrier()
    pltpu.sync_copy(acc.at[pl.ds(s * SLAB, SLAB)],
                    out_hbm.at[pl.ds(s * SLAB, SLAB)])


def _agg_body(s0, s1, s2, s3, packedR, zrows,
              o0, o1, o2, o3,
              packed_v, svring, dvring, b0, b1, acc,
              g0, g1, ss0, ss1):
    c = lax.axis_index("c")
    s = lax.axis_index("s")
    pltpu.sync_copy(packedR.at[pl.ds(s * AGG_ROWS, AGG_ROWS)], packed_v)
    s_in = (s0, s1, s2, s3)
    outs = (o0, o1, o2, o3)
    bufs = (b0, b1)
    gsems = (g0, g1)
    ssems = (ss0, ss1)
    for ch in range(4):
        pl.when((ch % 2) == c)(functools.partial(
            _agg_chunk, s_in[ch], outs[ch], zrows, packed_v,
            svring, dvring, bufs, gsems, ssems, acc, s))


def _agg_pallas(s_list, packedR, zrows):
    mesh = plsc.VectorSubcoreMesh(core_axis_name="c", subcore_axis_name="s")
    f = pl.kernel(
        _agg_body,
        out_type=[jax.ShapeDtypeStruct((NP, 128), jnp.float32)] * 4,
        mesh=mesh,
        scratch_types=[
            pltpu.VMEM((AGG_ROWS, 128), jnp.int32),
            pltpu.VMEM((NBUF, UNIT), jnp.int32),
            pltpu.VMEM((NBUF, UNIT), jnp.int32),
        ] + [pltpu.VMEM((UNIT, 128), jnp.float32)] * NBUF + [
            pltpu.VMEM_SHARED((NP, 128), jnp.float32),
        ] + [pltpu.SemaphoreType.DMA] * (2 * NBUF),
    )
    return f(*s_list, packedR, zrows)


def _maxk_mask(t):
    imin = jnp.int32(-(2 ** 31))
    imax = jnp.int32(2 ** 31 - 1)
    b = lax.bitcast_convert_type(t, jnp.int32)
    k = jnp.where(b >= 0, b, imin - b)
    kf = jnp.float32(K)
    cntpos = jnp.sum(jnp.where(k >= 0, 1.0, 0.0), axis=1, keepdims=True)
    possel = cntpos >= kf
    kk = jnp.where(k >= 0,
                   jnp.where(possel, k, imax),
                   jnp.where(possel, jnp.int32(-1), k - imin))
    T = jnp.zeros(possel.shape, jnp.int32)
    for i in range(30, -1, -1):
        cand = jnp.bitwise_or(T, jnp.int32(1 << i))
        cnt = jnp.sum(jnp.where(kk >= cand, 1.0, 0.0), axis=1, keepdims=True)
        T = jnp.where(cnt >= kf, cand, T)
    gt = kk > T
    eq = kk == T
    need = kf - jnp.sum(jnp.where(gt, 1.0, 0.0), axis=1, keepdims=True)
    lane = lax.broadcasted_iota(jnp.int32, t.shape, 1)
    X = jnp.zeros(possel.shape, jnp.int32)
    for i in range(9, -1, -1):
        cand = X + jnp.int32(1 << i)
        cnt = jnp.sum(jnp.where(jnp.logical_and(eq, lane < cand), 1.0, 0.0),
                      axis=1, keepdims=True)
        X = jnp.where(cnt <= need, cand, X)
    return jnp.logical_or(gt, jnp.logical_and(eq, lane < X))


def _finish_maxk(t, nsrc_ref, outs):
    mask = _maxk_mask(t)
    sc = jnp.where(mask, t, 0.0) * nsrc_ref[...]
    gid = pl.program_id(0) * BM + lax.broadcasted_iota(jnp.int32, (BM, 1), 0)
    sc = jnp.where(gid < N, sc, 0.0)
    for c4 in range(4):
        outs[c4][...] = sc[:, c4 * 128:(c4 + 1) * 128]


def _mm_in_body(x_ref, w_ref, b_ref, o_ref):
    o_ref[...] = jnp.maximum(
        jnp.dot(x_ref[...], w_ref[...], preferred_element_type=jnp.float32)
        + b_ref[...], 0.0)


def _mm_in(x, W, b):
    return pl.pallas_call(
        _mm_in_body,
        grid=(GRID_M,),
        in_specs=[
            pl.BlockSpec((BM, IN), lambda m: (m, 0)),
            pl.BlockSpec((IN, HID), lambda m: (0, 0)),
            pl.BlockSpec((1, HID), lambda m: (0, 0)),
        ],
        out_specs=pl.BlockSpec((BM, HID), lambda m: (m, 0)),
        out_shape=jax.ShapeDtypeStruct((NP, HID), jnp.float32),
    )(x, W, b)


def _mm_maxk_first_body(h_ref, w_ref, b_ref, nsrc_ref, o0, o1, o2, o3):
    t = jnp.dot(h_ref[...], w_ref[...], preferred_element_type=jnp.float32) \
        + b_ref[...]
    _finish_maxk(t, nsrc_ref, (o0, o1, o2, o3))


def _mm_maxk_first(h, W, b, nsrc):
    return pl.pallas_call(
        _mm_maxk_first_body,
        grid=(GRID_M,),
        in_specs=[
            pl.BlockSpec((BM, HID), lambda m: (m, 0)),
            pl.BlockSpec((HID, HID), lambda m: (0, 0)),
            pl.BlockSpec((1, HID), lambda m: (0, 0)),
            pl.BlockSpec((BM, 1), lambda m: (m, 0)),
        ],
        out_specs=[pl.BlockSpec((BM, 128), lambda m: (m, 0))] * 4,
        out_shape=[jax.ShapeDtypeStruct((NP, 128), jnp.float32)] * 4,
    )(h, W, b, nsrc)


def _gconv_in(a_refs, ndst_ref, gb_ref):
    hin = jnp.concatenate([a_refs[c][...] for c in range(4)], axis=1)
    return hin * ndst_ref[...] + gb_ref[...]


def _mm_maxk_next_body(a0, a1, a2, a3, ndst_ref, gb_ref, w_ref, b_ref,
                       nsrc_ref, o0, o1, o2, o3):
    hin = _gconv_in((a0, a1, a2, a3), ndst_ref, gb_ref)
    t = jnp.dot(hin, w_ref[...], preferred_element_type=jnp.float32) \
        + b_ref[...]
    _finish_maxk(t, nsrc_ref, (o0, o1, o2, o3))


def _mm_maxk_next(a_list, ndst, gb, W, b, nsrc):
    return pl.pallas_call(
        _mm_maxk_next_body,
        grid=(GRID_M,),
        in_specs=[pl.BlockSpec((BM, 128), lambda m: (m, 0))] * 4 + [
            pl.BlockSpec((BM, 1), lambda m: (m, 0)),
            pl.BlockSpec((1, HID), lambda m: (0, 0)),
            pl.BlockSpec((HID, HID), lambda m: (0, 0)),
            pl.BlockSpec((1, HID), lambda m: (0, 0)),
            pl.BlockSpec((BM, 1), lambda m: (m, 0)),
        ],
        out_specs=[pl.BlockSpec((BM, 128), lambda m: (m, 0))] * 4,
        out_shape=[jax.ShapeDtypeStruct((NP, 128), jnp.float32)] * 4,
    )(*a_list, ndst, gb, W, b, nsrc)


def _mm_out_body(a0, a1, a2, a3, ndst_ref, gb_ref, w_ref, b_ref, o_ref):
    hin = _gconv_in((a0, a1, a2, a3), ndst_ref, gb_ref)
    o_ref[...] = jnp.dot(hin, w_ref[...], preferred_element_type=jnp.float32) \
        + b_ref[...]


def _mm_out(a_list, ndst, gb, W, b):
    return pl.pallas_call(
        _mm_out_body,
        grid=(GRID_M,),
        in_specs=[pl.BlockSpec((BM, 128), lambda m: (m, 0))] * 4 + [
            pl.BlockSpec((BM, 1), lambda m: (m, 0)),
            pl.BlockSpec((1, HID), lambda m: (0, 0)),
            pl.BlockSpec((HID, OUT), lambda m: (0, 0)),
            pl.BlockSpec((1, OUT), lambda m: (0, 0)),
        ],
        out_specs=pl.BlockSpec((BM, OUT), lambda m: (m, 0)),
        out_shape=jax.ShapeDtypeStruct((NP, OUT), jnp.float32),
    )(*a_list, ndst, gb, W, b)


def kernel(x, edge_index, W_in, b_in, W_l0, b_l0, W_l1, b_l1, W_l2, b_l2,
           gb0, gb1, gb2, W_out, b_out):
    src = edge_index[0]
    dst = edge_index[1]
    pad = jnp.full((EP - E,), PAD_NODE, jnp.int32)
    srcR = jnp.concatenate([src, pad]).reshape(EROWS, 128)
    dstR = jnp.concatenate([dst, pad]).reshape(EROWS, 128)
    packedR = jnp.bitwise_or(srcR, jnp.left_shift(dstR, 16))
    x_pad = jnp.zeros((NP, IN), jnp.float32).at[:N].set(x)
    zbins = jnp.zeros((NP,), jnp.float32)
    zrows = jnp.zeros((SLAB, 128), jnp.float32)

    partials = _deg_pallas(srcR, dstR, zbins)
    deg = partials.sum(axis=0)
    norm = lax.rsqrt(jnp.maximum(deg, 1.0))
    nsrc = norm[0].reshape(NP, 1)
    ndst = norm[1].reshape(NP, 1)

    h = _mm_in(x_pad, W_in, b_in.reshape(1, HID))
    s_list = _mm_maxk_first(h, W_l0, b_l0.reshape(1, HID), nsrc)
    for (Wl, bl, gb) in ((W_l1, b_l1, gb0), (W_l2, b_l2, gb1)):
        a_list = _agg_pallas(s_list, packedR, zrows)
        s_list = _mm_maxk_next(a_list, ndst, gb.reshape(1, HID), Wl,
                               bl.reshape(1, HID), nsrc)
    a_list = _agg_pallas(s_list, packedR, zrows)
    out = _mm_out(a_list, ndst, gb2.reshape(1, HID), W_out,
                  b_out.reshape(1, OUT))
    return out[:N]


# speedup vs baseline: 1.0119x; 1.0119x over previous
import functools

import jax
import jax.numpy as jnp
from jax import lax
from jax.experimental import pallas as pl
from jax.experimental.pallas import tpu as pltpu
from jax.experimental.pallas import tpu_sc as plsc

N = 10000
NP = 10240
E = 160000
EP = 163840
IN = 256
HID = 512
OUT = 128
K = 32
PAD_NODE = 10200

BM = 512
GRID_M = NP // BM
NSUB = 16
NCORE = 2
EROWS = EP // 128
DEG_ROWS = EROWS // (NSUB * NCORE)
AGG_ROWS = EROWS // NSUB
SLAB = NP // NSUB


def _deg_body(srcR, dstR, zbins, out, sv, dv, bins_s, bins_d):
    c = lax.axis_index("c")
    s = lax.axis_index("s")
    w = s * NCORE + c
    pltpu.sync_copy(srcR.at[pl.ds(w * DEG_ROWS, DEG_ROWS)], sv)
    pltpu.sync_copy(dstR.at[pl.ds(w * DEG_ROWS, DEG_ROWS)], dv)
    pltpu.sync_copy(zbins, bins_s)
    pltpu.sync_copy(zbins, bins_d)
    ones = jnp.ones((16,), jnp.float32)

    def step(j, carry):
        r = j // 8
        off = (j % 8) * 16
        plsc.addupdate_scatter(bins_s, [sv[r, pl.ds(off, 16)]], ones)
        plsc.addupdate_scatter(bins_d, [dv[r, pl.ds(off, 16)]], ones)
        return carry

    lax.fori_loop(0, DEG_ROWS * 8, step, 0)
    pltpu.sync_copy(bins_s, out.at[w, 0])
    pltpu.sync_copy(bins_d, out.at[w, 1])


def _deg_pallas(srcR, dstR, zbins):
    mesh = plsc.VectorSubcoreMesh(core_axis_name="c", subcore_axis_name="s")
    f = pl.kernel(
        _deg_body,
        out_type=jax.ShapeDtypeStruct((NCORE * NSUB, 2, NP), jnp.float32),
        mesh=mesh,
        compiler_params=pltpu.CompilerParams(needs_layout_passes=False),
        scratch_types=[
            pltpu.VMEM((DEG_ROWS, 128), jnp.int32),
            pltpu.VMEM((DEG_ROWS, 128), jnp.int32),
            pltpu.VMEM((NP,), jnp.float32),
            pltpu.VMEM((NP,), jnp.float32),
        ],
    )
    return f(srcR, dstR, zbins)


NBUF = 4
UNIT = 64
UNITS = (AGG_ROWS * 128) // UNIT


def _extract(packed_v, svring, dvring, u, unit):
    for i in range(UNIT // 16):
        off = unit * UNIT + i * 16
        p = packed_v[off // 128, pl.ds(off % 128, 16)]
        svring[u, pl.ds(i * 16, 16)] = jnp.bitwise_and(p, 0xFFFF)
        dvring[u, pl.ds(i * 16, 16)] = lax.shift_right_logical(p, 16)


def _agg_chunk(s_hbm, out_hbm, zrows, packed_v, svring, dvring,
               bufs, gsems, ssems, acc, s):
    pltpu.sync_copy(zrows, acc.at[pl.ds(s * SLAB, SLAB)])
    plsc.subcore_barrier()
    for u in range(NBUF):
        _extract(packed_v, svring, dvring, u, u)
        pltpu.async_copy(s_hbm.at[svring.at[u]], bufs[u], gsems[u])

    def step(i, carry):
        j = i * NBUF
        for u in range(NBUF):
            jj = j + u
            pltpu.make_async_copy(s_hbm.at[svring.at[u]], bufs[u],
                                  gsems[u]).wait()
            pltpu.async_copy(bufs[u], acc.at[dvring.at[u]], ssems[u],
                             add=True)

            @pl.when(jj + NBUF < UNITS)
            def _(u=u, jj=jj):
                pltpu.make_async_copy(bufs[u], acc.at[dvring.at[u]],
                                      ssems[u]).wait()
                _extract(packed_v, svring, dvring, u, jj + NBUF)
                pltpu.async_copy(s_hbm.at[svring.at[u]], bufs[u],
                                 gsems[u])
        return carry

    lax.fori_loop(0, UNITS // NBUF, step, 0)
    for u in range(NBUF):
        pltpu.make_async_copy(bufs[u], acc.at[dvring.at[u]],
                              ssems[u]).wait()
    plsc.subcore_barrier()
    pltpu.sync_copy(acc.at[pl.ds(s * SLAB, SLAB)],
                    out_hbm.at[pl.ds(s * SLAB, SLAB)])


def _agg_body(s0, s1, s2, s3, packedR, zrows,
              o0, o1, o2, o3,
              packed_v, svring, dvring, b0, b1, b2, b3, acc,
              g0, g1, g2, g3, ss0, ss1, ss2, ss3):
    c = lax.axis_index("c")
    s = lax.axis_index("s")
    pltpu.sync_copy(packedR.at[pl.ds(s * AGG_ROWS, AGG_ROWS)], packed_v)
    s_in = (s0, s1, s2, s3)
    outs = (o0, o1, o2, o3)
    bufs = (b0, b1, b2, b3)
    gsems = (g0, g1, g2, g3)
    ssems = (ss0, ss1, ss2, ss3)
    for ch in range(4):
        pl.when((ch % 2) == c)(functools.partial(
            _agg_chunk, s_in[ch], outs[ch], zrows, packed_v,
            svring, dvring, bufs, gsems, ssems, acc, s))


def _agg_pallas(s_list, packedR, zrows):
    mesh = plsc.VectorSubcoreMesh(core_axis_name="c", subcore_axis_name="s")
    f = pl.kernel(
        _agg_body,
        out_type=[jax.ShapeDtypeStruct((NP, 128), jnp.float32)] * 4,
        mesh=mesh,
        scratch_types=[
            pltpu.VMEM((AGG_ROWS, 128), jnp.int32),
            pltpu.VMEM((NBUF, UNIT), jnp.int32),
            pltpu.VMEM((NBUF, UNIT), jnp.int32),
        ] + [pltpu.VMEM((UNIT, 128), jnp.float32)] * NBUF + [
            pltpu.VMEM_SHARED((NP, 128), jnp.float32),
        ] + [pltpu.SemaphoreType.DMA] * (2 * NBUF),
    )
    return f(*s_list, packedR, zrows)


def _maxk_mask(t):
    imin = jnp.int32(-(2 ** 31))
    imax = jnp.int32(2 ** 31 - 1)
    b = lax.bitcast_convert_type(t, jnp.int32)
    k = jnp.where(b >= 0, b, imin - b)
    kf = jnp.float32(K)
    cntpos = jnp.sum(jnp.where(k >= 0, 1.0, 0.0), axis=1, keepdims=True)
    possel = cntpos >= kf
    kk = jnp.where(k >= 0,
                   jnp.where(possel, k, imax),
                   jnp.where(possel, jnp.int32(-1), k - imin))
    T = jnp.zeros(possel.shape, jnp.int32)
    for i in range(30, -1, -1):
        cand = jnp.bitwise_or(T, jnp.int32(1 << i))
        cnt = jnp.sum(jnp.where(kk >= cand, 1.0, 0.0), axis=1, keepdims=True)
        T = jnp.where(cnt >= kf, cand, T)
    gt = kk > T
    eq = kk == T
    need = kf - jnp.sum(jnp.where(gt, 1.0, 0.0), axis=1, keepdims=True)
    lane = lax.broadcasted_iota(jnp.int32, t.shape, 1)
    X = jnp.zeros(possel.shape, jnp.int32)
    for i in range(9, -1, -1):
        cand = X + jnp.int32(1 << i)
        cnt = jnp.sum(jnp.where(jnp.logical_and(eq, lane < cand), 1.0, 0.0),
                      axis=1, keepdims=True)
        X = jnp.where(cnt <= need, cand, X)
    return jnp.logical_or(gt, jnp.logical_and(eq, lane < X))


def _finish_maxk(t, nsrc_ref, outs):
    mask = _maxk_mask(t)
    sc = jnp.where(mask, t, 0.0) * nsrc_ref[...]
    gid = pl.program_id(0) * BM + lax.broadcasted_iota(jnp.int32, (BM, 1), 0)
    sc = jnp.where(gid < N, sc, 0.0)
    for c4 in range(4):
        outs[c4][...] = sc[:, c4 * 128:(c4 + 1) * 128]


def _mm_in_body(x_ref, w_ref, b_ref, o_ref):
    o_ref[...] = jnp.maximum(
        jnp.dot(x_ref[...], w_ref[...], preferred_element_type=jnp.float32)
        + b_ref[...], 0.0)


def _mm_in(x, W, b):
    return pl.pallas_call(
        _mm_in_body,
        grid=(GRID_M,),
        in_specs=[
            pl.BlockSpec((BM, IN), lambda m: (m, 0)),
            pl.BlockSpec((IN, HID), lambda m: (0, 0)),
            pl.BlockSpec((1, HID), lambda m: (0, 0)),
        ],
        out_specs=pl.BlockSpec((BM, HID), lambda m: (m, 0)),
        out_shape=jax.ShapeDtypeStruct((NP, HID), jnp.float32),
    )(x, W, b)


def _mm_maxk_first_body(h_ref, w_ref, b_ref, nsrc_ref, o0, o1, o2, o3):
    t = jnp.dot(h_ref[...], w_ref[...], preferred_element_type=jnp.float32) \
        + b_ref[...]
    _finish_maxk(t, nsrc_ref, (o0, o1, o2, o3))


def _mm_maxk_first(h, W, b, nsrc):
    return pl.pallas_call(
        _mm_maxk_first_body,
        grid=(GRID_M,),
        in_specs=[
            pl.BlockSpec((BM, HID), lambda m: (m, 0)),
            pl.BlockSpec((HID, HID), lambda m: (0, 0)),
            pl.BlockSpec((1, HID), lambda m: (0, 0)),
            pl.BlockSpec((BM, 1), lambda m: (m, 0)),
        ],
        out_specs=[pl.BlockSpec((BM, 128), lambda m: (m, 0))] * 4,
        out_shape=[jax.ShapeDtypeStruct((NP, 128), jnp.float32)] * 4,
    )(h, W, b, nsrc)


def _gconv_in(a_refs, ndst_ref, gb_ref):
    hin = jnp.concatenate([a_refs[c][...] for c in range(4)], axis=1)
    return hin * ndst_ref[...] + gb_ref[...]


def _mm_maxk_next_body(a0, a1, a2, a3, ndst_ref, gb_ref, w_ref, b_ref,
                       nsrc_ref, o0, o1, o2, o3):
    hin = _gconv_in((a0, a1, a2, a3), ndst_ref, gb_ref)
    t = jnp.dot(hin, w_ref[...], preferred_element_type=jnp.float32) \
        + b_ref[...]
    _finish_maxk(t, nsrc_ref, (o0, o1, o2, o3))


def _mm_maxk_next(a_list, ndst, gb, W, b, nsrc):
    return pl.pallas_call(
        _mm_maxk_next_body,
        grid=(GRID_M,),
        in_specs=[pl.BlockSpec((BM, 128), lambda m: (m, 0))] * 4 + [
            pl.BlockSpec((BM, 1), lambda m: (m, 0)),
            pl.BlockSpec((1, HID), lambda m: (0, 0)),
            pl.BlockSpec((HID, HID), lambda m: (0, 0)),
            pl.BlockSpec((1, HID), lambda m: (0, 0)),
            pl.BlockSpec((BM, 1), lambda m: (m, 0)),
        ],
        out_specs=[pl.BlockSpec((BM, 128), lambda m: (m, 0))] * 4,
        out_shape=[jax.ShapeDtypeStruct((NP, 128), jnp.float32)] * 4,
    )(*a_list, ndst, gb, W, b, nsrc)


def _mm_out_body(a0, a1, a2, a3, ndst_ref, gb_ref, w_ref, b_ref, o_ref):
    hin = _gconv_in((a0, a1, a2, a3), ndst_ref, gb_ref)
    o_ref[...] = jnp.dot(hin, w_ref[...], preferred_element_type=jnp.float32) \
        + b_ref[...]


def _mm_out(a_list, ndst, gb, W, b):
    return pl.pallas_call(
        _mm_out_body,
        grid=(GRID_M,),
        in_specs=[pl.BlockSpec((BM, 128), lambda m: (m, 0))] * 4 + [
            pl.BlockSpec((BM, 1), lambda m: (m, 0)),
            pl.BlockSpec((1, HID), lambda m: (0, 0)),
            pl.BlockSpec((HID, OUT), lambda m: (0, 0)),
            pl.BlockSpec((1, OUT), lambda m: (0, 0)),
        ],
        out_specs=pl.BlockSpec((BM, OUT), lambda m: (m, 0)),
        out_shape=jax.ShapeDtypeStruct((NP, OUT), jnp.float32),
    )(*a_list, ndst, gb, W, b)


def kernel(x, edge_index, W_in, b_in, W_l0, b_l0, W_l1, b_l1, W_l2, b_l2,
           gb0, gb1, gb2, W_out, b_out):
    src = edge_index[0]
    dst = edge_index[1]
    pad = jnp.full((EP - E,), PAD_NODE, jnp.int32)
    srcR = jnp.concatenate([src, pad]).reshape(EROWS, 128)
    dstR = jnp.concatenate([dst, pad]).reshape(EROWS, 128)
    packedR = jnp.bitwise_or(srcR, jnp.left_shift(dstR, 16))
    x_pad = jnp.zeros((NP, IN), jnp.float32).at[:N].set(x)
    zbins = jnp.zeros((NP,), jnp.float32)
    zrows = jnp.zeros((SLAB, 128), jnp.float32)

    partials = _deg_pallas(srcR, dstR, zbins)
    deg = partials.sum(axis=0)
    norm = lax.rsqrt(jnp.maximum(deg, 1.0))
    nsrc = norm[0].reshape(NP, 1)
    ndst = norm[1].reshape(NP, 1)

    h = _mm_in(x_pad, W_in, b_in.reshape(1, HID))
    s_list = _mm_maxk_first(h, W_l0, b_l0.reshape(1, HID), nsrc)
    for (Wl, bl, gb) in ((W_l1, b_l1, gb0), (W_l2, b_l2, gb1)):
        a_list = _agg_pallas(s_list, packedR, zrows)
        s_list = _mm_maxk_next(a_list, ndst, gb.reshape(1, HID), Wl,
                               bl.reshape(1, HID), nsrc)
    a_list = _agg_pallas(s_list, packedR, zrows)
    out = _mm_out(a_list, ndst, gb2.reshape(1, HID), W_out,
                  b_out.reshape(1, OUT))
    return out[:N]
